# baseline (speedup 1.0000x reference)
import functools

import jax
import jax.numpy as jnp
from jax import lax
from jax.experimental import pallas as pl
from jax.experimental.pallas import tpu as pltpu
from jax.experimental.pallas import tpu_sc as plsc

T = 4096
D = 2048
H = 1024
E = 16
K = 2
SH = 1024
CAP = (T * K // E) * 2
TILE = 128
NP = (T * K + E * (TILE - 1) + TILE - 1) // TILE
PAD = NP * TILE
GATE_TT = 512
SHARED_TT = 512
NW = 32
SC_CHUNK = 32


def _gate_body(x_ref, gw_ref, idx_ref, w_ref):
    xb = x_ref[...]
    logits = lax.dot_general(xb, gw_ref[...], (((1,), (1,)), ((), ())),
                             preferred_element_type=jnp.float32)
    m = jnp.max(logits, axis=1, keepdims=True)
    ex = jnp.exp(logits - m)
    probs = ex / jnp.sum(ex, axis=1, keepdims=True)
    lanes = lax.broadcasted_iota(jnp.int32, logits.shape, 1)
    i1 = jnp.min(jnp.where(logits == m, lanes, E), axis=1, keepdims=True)
    l2 = jnp.where(lanes == i1, -jnp.inf, logits)
    m2 = jnp.max(l2, axis=1, keepdims=True)
    i2 = jnp.min(jnp.where(l2 == m2, lanes, E), axis=1, keepdims=True)
    w1v = jnp.sum(jnp.where(lanes == i1, probs, 0.0), axis=1, keepdims=True)
    w2v = jnp.sum(jnp.where(lanes == i2, probs, 0.0), axis=1, keepdims=True)
    idx_ref[...] = jnp.concatenate([i1, i2], axis=1)
    w_ref[...] = jnp.concatenate([w1v, w2v], axis=1)


def _gate(x, gate_w):
    return pl.pallas_call(
        _gate_body,
        grid=(T // GATE_TT,),
        in_specs=[
            pl.BlockSpec((GATE_TT, D), lambda i: (i, 0)),
            pl.BlockSpec((E, D), lambda i: (0, 0)),
        ],
        out_specs=[
            pl.BlockSpec((GATE_TT, K), lambda i: (i, 0)),
            pl.BlockSpec((GATE_TT, K), lambda i: (i, 0)),
        ],
        out_shape=[
            jax.ShapeDtypeStruct((T, K), jnp.int32),
            jax.ShapeDtypeStruct((T, K), jnp.float32),
        ],
    )(x, gate_w)


@functools.lru_cache(maxsize=None)
def _make_sc_gather(n_rows):
    b_per_w = n_rows // NW
    n_chunks = b_per_w // SC_CHUNK
    mesh = plsc.VectorSubcoreMesh(core_axis_name="c", subcore_axis_name="s")

    @functools.partial(
        pl.kernel, mesh=mesh,
        out_type=jax.ShapeDtypeStruct((n_rows, D), jnp.float32),
        scratch_types=[
            pltpu.VMEM((b_per_w,), jnp.int32),
            pltpu.VMEM((SC_CHUNK, D), jnp.float32),
            pltpu.SemaphoreType.DMA,
        ],
    )
    def k(table_hbm, idx_hbm, out_hbm, idx_v, rows_v, sem):
        wid = lax.axis_index("s") * 2 + lax.axis_index("c")
        base = wid * b_per_w
        pltpu.sync_copy(idx_hbm.at[pl.ds(base, b_per_w)], idx_v)
        for c in range(n_chunks):
            pltpu.async_copy(
                table_hbm.at[idx_v.at[pl.ds(c * SC_CHUNK, SC_CHUNK)]],
                rows_v, sem).wait()
            pltpu.sync_copy(rows_v, out_hbm.at[pl.ds(base + c * SC_CHUNK,
                                                     SC_CHUNK)])
    return k


def _dispatch_gather(table, idx):
    return _make_sc_gather(PAD)(table, idx)


def _combine_gather(table, idx):
    return _make_sc_gather(T * K)(table, idx)


def _ffn_body(te_ref, xs_ref, w1_ref, w3_ref, w2_ref, wr_ref, out_ref):
    del te_ref
    xb = xs_ref[...].astype(jnp.bfloat16)
    h = lax.dot_general(xb, w1_ref[0], (((1,), (1,)), ((), ())),
                        preferred_element_type=jnp.float32)
    u = lax.dot_general(xb, w3_ref[0], (((1,), (1,)), ((), ())),
                        preferred_element_type=jnp.float32)
    act = (h * jax.nn.sigmoid(h) * u).astype(jnp.bfloat16)
    o = lax.dot_general(act, w2_ref[0], (((1,), (1,)), ((), ())),
                        preferred_element_type=jnp.float32)
    out_ref[...] = o * wr_ref[...]


def _ffn(tile_expert, xs, w1, w3, w2, w_rows):
    grid_spec = pltpu.PrefetchScalarGridSpec(
        num_scalar_prefetch=1,
        grid=(NP,),
        in_specs=[
            pl.BlockSpec((TILE, D), lambda i, te: (i, 0)),
            pl.BlockSpec((1, H, D), lambda i, te: (te[i], 0, 0)),
            pl.BlockSpec((1, H, D), lambda i, te: (te[i], 0, 0)),
            pl.BlockSpec((1, D, H), lambda i, te: (te[i], 0, 0)),
            pl.BlockSpec((TILE, 1), lambda i, te: (i, 0)),
        ],
        out_specs=pl.BlockSpec((TILE, D), lambda i, te: (i, 0)),
    )
    return pl.pallas_call(
        _ffn_body,
        grid_spec=grid_spec,
        out_shape=jax.ShapeDtypeStruct((PAD, D), jnp.float32),
    )(tile_expert, xs, w1, w3, w2, w_rows)


def _shared_body(x_ref, sw1_ref, sw3_ref, sw2_ref, g_ref, out_ref):
    xb = x_ref[...].astype(jnp.bfloat16)
    h = lax.dot_general(xb, sw1_ref[...], (((1,), (1,)), ((), ())),
                        preferred_element_type=jnp.float32)
    u = lax.dot_general(xb, sw3_ref[...], (((1,), (1,)), ((), ())),
                        preferred_element_type=jnp.float32)
    act = (h * jax.nn.sigmoid(h) * u).astype(jnp.bfloat16)
    y = lax.dot_general(act, sw2_ref[...], (((1,), (1,)), ((), ())),
                        preferred_element_type=jnp.float32)
    g = g_ref[...]
    out_ref[...] = y + g[:, :D] + g[:, D:]


def _shared_combine(x, sw1, sw3, sw2, g2):
    return pl.pallas_call(
        _shared_body,
        grid=(T // SHARED_TT,),
        in_specs=[
            pl.BlockSpec((SHARED_TT, D), lambda i: (i, 0)),
            pl.BlockSpec((SH, D), lambda i: (0, 0)),
            pl.BlockSpec((SH, D), lambda i: (0, 0)),
            pl.BlockSpec((D, SH), lambda i: (0, 0)),
            pl.BlockSpec((SHARED_TT, 2 * D), lambda i: (i, 0)),
        ],
        out_specs=pl.BlockSpec((SHARED_TT, D), lambda i: (i, 0)),
        out_shape=jax.ShapeDtypeStruct((T, D), jnp.float32),
    )(x, sw1, sw3, sw2, g2)


def _routing_metadata(idx, wts):
    flat_e = idx.reshape(-1)
    oh = (flat_e[:, None] == jnp.arange(E, dtype=jnp.int32)[None, :])
    ohi = oh.astype(jnp.int32)
    pos = jnp.sum(jnp.cumsum(ohi, axis=0) * ohi, axis=1) - 1
    counts = jnp.sum(ohi, axis=0)
    kept = jnp.minimum(counts, CAP)
    padded = ((kept + TILE - 1) // TILE) * TILE
    ends = jnp.cumsum(padded)
    offs = ends - padded
    keep = pos < CAP
    dest = jnp.where(keep, offs[flat_e] + pos, PAD - 1)
    tok = (jnp.arange(T * K, dtype=jnp.int32) // K)
    src = jnp.full((PAD,), T, jnp.int32).at[dest].set(tok)
    wflat = wts.reshape(-1) * keep.astype(jnp.float32)
    w_rows = jnp.zeros((PAD, 1), jnp.float32).at[dest, 0].set(wflat)
    w_rows = w_rows.at[PAD - 1, 0].set(0.0)
    tile_start = jnp.arange(NP, dtype=jnp.int32) * TILE
    te = jnp.sum((tile_start[:, None] >= ends[None, :]).astype(jnp.int32),
                 axis=1)
    te = jnp.minimum(te, E - 1)
    return dest, src, w_rows, te


def kernel(x, gate_w, w1, w3, w2, sw1, sw3, sw2):
    idx, wts = _gate(x, gate_w)
    dest, src, w_rows, te = _routing_metadata(idx, wts)
    x_aug = jnp.concatenate([x, jnp.zeros((8, D), x.dtype)], axis=0)
    xs = _dispatch_gather(x_aug, src)
    bf = jnp.bfloat16
    outbuf = _ffn(te, xs, w1.astype(bf), w3.astype(bf), w2.astype(bf),
                  w_rows)
    g = _combine_gather(outbuf, dest)
    y = _shared_combine(x, sw1.astype(bf), sw3.astype(bf), sw2.astype(bf),
                        g.reshape(T, K * D))
    return y


# speedup vs baseline: 1.1345x; 1.1345x over previous
import functools

import jax
import jax.numpy as jnp
from jax import lax
from jax.experimental import pallas as pl
from jax.experimental.pallas import tpu as pltpu
from jax.experimental.pallas import tpu_sc as plsc

T = 4096
D = 2048
H = 1024
E = 16
K = 2
SH = 1024
CAP = (T * K // E) * 2
TILE = 128
NP = (T * K + E * (TILE - 1) + TILE - 1) // TILE
PAD = NP * TILE
GATE_TT = 512
SHARED_TT = 512
NW = 32
SC_CHUNK = 16


def _gate_body(x_ref, gw_ref, idx_ref, w_ref):
    xb = x_ref[...]
    logits = lax.dot_general(xb, gw_ref[...], (((1,), (1,)), ((), ())),
                             preferred_element_type=jnp.float32)
    m = jnp.max(logits, axis=1, keepdims=True)
    ex = jnp.exp(logits - m)
    probs = ex / jnp.sum(ex, axis=1, keepdims=True)
    lanes = lax.broadcasted_iota(jnp.int32, logits.shape, 1)
    i1 = jnp.min(jnp.where(logits == m, lanes, E), axis=1, keepdims=True)
    l2 = jnp.where(lanes == i1, -jnp.inf, logits)
    m2 = jnp.max(l2, axis=1, keepdims=True)
    i2 = jnp.min(jnp.where(l2 == m2, lanes, E), axis=1, keepdims=True)
    w1v = jnp.sum(jnp.where(lanes == i1, probs, 0.0), axis=1, keepdims=True)
    w2v = jnp.sum(jnp.where(lanes == i2, probs, 0.0), axis=1, keepdims=True)
    idx_ref[...] = jnp.concatenate([i1, i2], axis=1)
    w_ref[...] = jnp.concatenate([w1v, w2v], axis=1)


def _gate(x, gate_w):
    return pl.pallas_call(
        _gate_body,
        grid=(T // GATE_TT,),
        in_specs=[
            pl.BlockSpec((GATE_TT, D), lambda i: (i, 0)),
            pl.BlockSpec((E, D), lambda i: (0, 0)),
        ],
        out_specs=[
            pl.BlockSpec((GATE_TT, K), lambda i: (i, 0)),
            pl.BlockSpec((GATE_TT, K), lambda i: (i, 0)),
        ],
        out_shape=[
            jax.ShapeDtypeStruct((T, K), jnp.int32),
            jax.ShapeDtypeStruct((T, K), jnp.float32),
        ],
    )(x, gate_w)


@functools.lru_cache(maxsize=None)
def _make_sc_gather(n_rows):
    b_per_w = n_rows // NW
    n_chunks = b_per_w // SC_CHUNK
    mesh = plsc.VectorSubcoreMesh(core_axis_name="c", subcore_axis_name="s")

    @functools.partial(
        pl.kernel, mesh=mesh,
        out_type=jax.ShapeDtypeStruct((n_rows, D), jnp.float32),
        scratch_types=[
            pltpu.VMEM((b_per_w,), jnp.int32),
            pltpu.VMEM((SC_CHUNK, D), jnp.float32),
            pltpu.VMEM((SC_CHUNK, D), jnp.float32),
            pltpu.SemaphoreType.DMA,
            pltpu.SemaphoreType.DMA,
        ],
    )
    def k(table_hbm, idx_hbm, out_hbm, idx_v, rows0, rows1, sem0, sem1):
        wid = lax.axis_index("s") * 2 + lax.axis_index("c")
        base = wid * b_per_w
        pltpu.sync_copy(idx_hbm.at[pl.ds(base, b_per_w)], idx_v)
        bufs = (rows0, rows1)
        sems = (sem0, sem1)

        def gather(c):
            return pltpu.async_copy(
                table_hbm.at[idx_v.at[pl.ds(c * SC_CHUNK, SC_CHUNK)]],
                bufs[c % 2], sems[c % 2])

        handles = {0: gather(0)}
        for c in range(n_chunks):
            if c + 1 < n_chunks:
                handles[c + 1] = gather(c + 1)
            handles[c].wait()
            pltpu.sync_copy(bufs[c % 2],
                            out_hbm.at[pl.ds(base + c * SC_CHUNK, SC_CHUNK)])
    return k


def _dispatch_gather(table, idx):
    return _make_sc_gather(PAD)(table, idx)


def _combine_gather(table, idx):
    return _make_sc_gather(T * K)(table, idx)


def _ffn_body(te_ref, xs_ref, w1_ref, w3_ref, w2_ref, wr_ref, out_ref):
    del te_ref
    xb = xs_ref[...]
    h = lax.dot_general(xb, w1_ref[0], (((1,), (1,)), ((), ())),
                        preferred_element_type=jnp.float32)
    u = lax.dot_general(xb, w3_ref[0], (((1,), (1,)), ((), ())),
                        preferred_element_type=jnp.float32)
    act = h * jax.nn.sigmoid(h) * u
    o = lax.dot_general(act, w2_ref[0], (((1,), (1,)), ((), ())),
                        preferred_element_type=jnp.float32)
    out_ref[...] = o * wr_ref[...]


def _ffn(tile_expert, xs, w1, w3, w2, w_rows):
    grid_spec = pltpu.PrefetchScalarGridSpec(
        num_scalar_prefetch=1,
        grid=(NP,),
        in_specs=[
            pl.BlockSpec((TILE, D), lambda i, te: (i, 0)),
            pl.BlockSpec((1, H, D), lambda i, te: (te[i], 0, 0)),
            pl.BlockSpec((1, H, D), lambda i, te: (te[i], 0, 0)),
            pl.BlockSpec((1, D, H), lambda i, te: (te[i], 0, 0)),
            pl.BlockSpec((TILE, 1), lambda i, te: (i, 0)),
        ],
        out_specs=pl.BlockSpec((TILE, D), lambda i, te: (i, 0)),
    )
    return pl.pallas_call(
        _ffn_body,
        grid_spec=grid_spec,
        out_shape=jax.ShapeDtypeStruct((PAD, D), jnp.float32),
    )(tile_expert, xs, w1, w3, w2, w_rows)


def _shared_body(x_ref, sw1_ref, sw3_ref, sw2_ref, g_ref, out_ref):
    xb = x_ref[...]
    h = lax.dot_general(xb, sw1_ref[...], (((1,), (1,)), ((), ())),
                        preferred_element_type=jnp.float32)
    u = lax.dot_general(xb, sw3_ref[...], (((1,), (1,)), ((), ())),
                        preferred_element_type=jnp.float32)
    act = h * jax.nn.sigmoid(h) * u
    y = lax.dot_general(act, sw2_ref[...], (((1,), (1,)), ((), ())),
                        preferred_element_type=jnp.float32)
    g = g_ref[...]
    out_ref[...] = y + g[:, :D] + g[:, D:]


def _shared_combine(x, sw1, sw3, sw2, g2):
    return pl.pallas_call(
        _shared_body,
        grid=(T // SHARED_TT,),
        in_specs=[
            pl.BlockSpec((SHARED_TT, D), lambda i: (i, 0)),
            pl.BlockSpec((SH, D), lambda i: (0, 0)),
            pl.BlockSpec((SH, D), lambda i: (0, 0)),
            pl.BlockSpec((D, SH), lambda i: (0, 0)),
            pl.BlockSpec((SHARED_TT, 2 * D), lambda i: (i, 0)),
        ],
        out_specs=pl.BlockSpec((SHARED_TT, D), lambda i: (i, 0)),
        out_shape=jax.ShapeDtypeStruct((T, D), jnp.float32),
    )(x, sw1, sw3, sw2, g2)


def _routing_metadata(idx, wts):
    flat_e = idx.reshape(-1)
    oh = (flat_e[:, None] == jnp.arange(E, dtype=jnp.int32)[None, :])
    ohi = oh.astype(jnp.int32)
    pos = jnp.sum(jnp.cumsum(ohi, axis=0) * ohi, axis=1) - 1
    counts = jnp.sum(ohi, axis=0)
    kept = jnp.minimum(counts, CAP)
    padded = ((kept + TILE - 1) // TILE) * TILE
    ends = jnp.cumsum(padded)
    offs = ends - padded
    keep = pos < CAP
    dest = jnp.where(keep, offs[flat_e] + pos, PAD - 1)
    tok = (jnp.arange(T * K, dtype=jnp.int32) // K)
    src = jnp.full((PAD,), T, jnp.int32).at[dest].set(tok)
    wflat = wts.reshape(-1) * keep.astype(jnp.float32)
    w_rows = jnp.zeros((PAD, 1), jnp.float32).at[dest, 0].set(wflat)
    w_rows = w_rows.at[PAD - 1, 0].set(0.0)
    tile_start = jnp.arange(NP, dtype=jnp.int32) * TILE
    te = jnp.sum((tile_start[:, None] >= ends[None, :]).astype(jnp.int32),
                 axis=1)
    te = jnp.minimum(te, E - 1)
    return dest, src, w_rows, te


def kernel(x, gate_w, w1, w3, w2, sw1, sw3, sw2):
    idx, wts = _gate(x, gate_w)
    dest, src, w_rows, te = _routing_metadata(idx, wts)
    x_aug = jnp.concatenate([x, jnp.zeros((8, D), x.dtype)], axis=0)
    xs = _dispatch_gather(x_aug, src)
    outbuf = _ffn(te, xs, w1, w3, w2, w_rows)
    g = _combine_gather(outbuf, dest)
    y = _shared_combine(x, sw1, sw3, sw2, g.reshape(T, K * D))
    return y


# speedup vs baseline: 1.1484x; 1.0123x over previous
import functools

import jax
import jax.numpy as jnp
from jax import lax
from jax.experimental import pallas as pl
from jax.experimental.pallas import tpu as pltpu
from jax.experimental.pallas import tpu_sc as plsc

T = 4096
D = 2048
H = 1024
E = 16
K = 2
SH = 1024
CAP = (T * K // E) * 2
TILE = 128
NP = (T * K + E * (TILE - 1) + TILE - 1) // TILE
PAD = NP * TILE
GATE_TT = 512
SHARED_TT = 512
NW = 32
SC_CHUNK = 16


def _gate_body(x_ref, gw_ref, idx_ref, w_ref):
    xb = x_ref[...]
    logits = lax.dot_general(xb, gw_ref[...], (((1,), (1,)), ((), ())),
                             preferred_element_type=jnp.float32)
    m = jnp.max(logits, axis=1, keepdims=True)
    ex = jnp.exp(logits - m)
    probs = ex / jnp.sum(ex, axis=1, keepdims=True)
    lanes = lax.broadcasted_iota(jnp.int32, logits.shape, 1)
    i1 = jnp.min(jnp.where(logits == m, lanes, E), axis=1, keepdims=True)
    l2 = jnp.where(lanes == i1, -jnp.inf, logits)
    m2 = jnp.max(l2, axis=1, keepdims=True)
    i2 = jnp.min(jnp.where(l2 == m2, lanes, E), axis=1, keepdims=True)
    w1v = jnp.sum(jnp.where(lanes == i1, probs, 0.0), axis=1, keepdims=True)
    w2v = jnp.sum(jnp.where(lanes == i2, probs, 0.0), axis=1, keepdims=True)
    idx_ref[...] = jnp.concatenate([i1, i2], axis=1)
    w_ref[...] = jnp.concatenate([w1v, w2v], axis=1)


def _gate(x, gate_w):
    return pl.pallas_call(
        _gate_body,
        grid=(T // GATE_TT,),
        in_specs=[
            pl.BlockSpec((GATE_TT, D), lambda i: (i, 0)),
            pl.BlockSpec((E, D), lambda i: (0, 0)),
        ],
        out_specs=[
            pl.BlockSpec((GATE_TT, K), lambda i: (i, 0)),
            pl.BlockSpec((GATE_TT, K), lambda i: (i, 0)),
        ],
        out_shape=[
            jax.ShapeDtypeStruct((T, K), jnp.int32),
            jax.ShapeDtypeStruct((T, K), jnp.float32),
        ],
    )(x, gate_w)


@functools.lru_cache(maxsize=None)
def _make_sc_gather(n_rows):
    n_chunks = n_rows // (NW * SC_CHUNK)
    mesh = plsc.VectorSubcoreMesh(core_axis_name="c", subcore_axis_name="s")

    @functools.partial(
        pl.kernel, mesh=mesh,
        out_type=jax.ShapeDtypeStruct((n_rows, D), jnp.float32),
        scratch_types=[
            pltpu.VMEM((n_rows,), jnp.int32),
            pltpu.VMEM((SC_CHUNK, D), jnp.float32),
            pltpu.VMEM((SC_CHUNK, D), jnp.float32),
            pltpu.SemaphoreType.DMA,
            pltpu.SemaphoreType.DMA,
        ],
    )
    def k(table_hbm, idx_hbm, out_hbm, idx_v, rows0, rows1, sem0, sem1):
        wid = lax.axis_index("s") * 2 + lax.axis_index("c")
        pltpu.sync_copy(idx_hbm, idx_v)
        bufs = (rows0, rows1)
        sems = (sem0, sem1)

        def row0(c):
            return (wid + c * NW) * SC_CHUNK

        def gather(c):
            return pltpu.async_copy(
                table_hbm.at[idx_v.at[pl.ds(row0(c), SC_CHUNK)]],
                bufs[c % 2], sems[c % 2])

        handles = {0: gather(0)}
        for c in range(n_chunks):
            if c + 1 < n_chunks:
                handles[c + 1] = gather(c + 1)
            handles[c].wait()
            pltpu.sync_copy(bufs[c % 2],
                            out_hbm.at[pl.ds(row0(c), SC_CHUNK)])
    return k


def _dispatch_gather(table, idx):
    return _make_sc_gather(PAD)(table, idx)


def _combine_gather(table, idx):
    return _make_sc_gather(T * K)(table, idx)


def _ffn_body(te_ref, xs_ref, w1_ref, w3_ref, w2_ref, wr_ref, out_ref):
    del te_ref
    xb = xs_ref[...]
    h = lax.dot_general(xb, w1_ref[0], (((1,), (1,)), ((), ())),
                        preferred_element_type=jnp.float32)
    u = lax.dot_general(xb, w3_ref[0], (((1,), (1,)), ((), ())),
                        preferred_element_type=jnp.float32)
    act = h * jax.nn.sigmoid(h) * u
    o = lax.dot_general(act, w2_ref[0], (((1,), (1,)), ((), ())),
                        preferred_element_type=jnp.float32)
    out_ref[...] = o * wr_ref[...]


def _ffn(tile_expert, xs, w1, w3, w2, w_rows):
    grid_spec = pltpu.PrefetchScalarGridSpec(
        num_scalar_prefetch=1,
        grid=(NP,),
        in_specs=[
            pl.BlockSpec((TILE, D), lambda i, te: (i, 0)),
            pl.BlockSpec((1, H, D), lambda i, te: (te[i], 0, 0)),
            pl.BlockSpec((1, H, D), lambda i, te: (te[i], 0, 0)),
            pl.BlockSpec((1, D, H), lambda i, te: (te[i], 0, 0)),
            pl.BlockSpec((TILE, 1), lambda i, te: (i, 0)),
        ],
        out_specs=pl.BlockSpec((TILE, D), lambda i, te: (i, 0)),
    )
    return pl.pallas_call(
        _ffn_body,
        grid_spec=grid_spec,
        out_shape=jax.ShapeDtypeStruct((PAD, D), jnp.float32),
    )(tile_expert, xs, w1, w3, w2, w_rows)


def _shared_body(x_ref, sw1_ref, sw3_ref, sw2_ref, g_ref, out_ref):
    xb = x_ref[...]
    h = lax.dot_general(xb, sw1_ref[...], (((1,), (1,)), ((), ())),
                        preferred_element_type=jnp.float32)
    u = lax.dot_general(xb, sw3_ref[...], (((1,), (1,)), ((), ())),
                        preferred_element_type=jnp.float32)
    act = h * jax.nn.sigmoid(h) * u
    y = lax.dot_general(act, sw2_ref[...], (((1,), (1,)), ((), ())),
                        preferred_element_type=jnp.float32)
    g = g_ref[...]
    out_ref[...] = y + g[:, :D] + g[:, D:]


def _shared_combine(x, sw1, sw3, sw2, g2):
    return pl.pallas_call(
        _shared_body,
        grid=(T // SHARED_TT,),
        in_specs=[
            pl.BlockSpec((SHARED_TT, D), lambda i: (i, 0)),
            pl.BlockSpec((SH, D), lambda i: (0, 0)),
            pl.BlockSpec((SH, D), lambda i: (0, 0)),
            pl.BlockSpec((D, SH), lambda i: (0, 0)),
            pl.BlockSpec((SHARED_TT, 2 * D), lambda i: (i, 0)),
        ],
        out_specs=pl.BlockSpec((SHARED_TT, D), lambda i: (i, 0)),
        out_shape=jax.ShapeDtypeStruct((T, D), jnp.float32),
    )(x, sw1, sw3, sw2, g2)


def _routing_metadata(idx, wts):
    flat_e = idx.reshape(-1)
    oh = (flat_e[:, None] == jnp.arange(E, dtype=jnp.int32)[None, :])
    ohi = oh.astype(jnp.int32)
    pos = jnp.sum(jnp.cumsum(ohi, axis=0) * ohi, axis=1) - 1
    counts = jnp.sum(ohi, axis=0)
    kept = jnp.minimum(counts, CAP)
    padded = ((kept + TILE - 1) // TILE) * TILE
    ends = jnp.cumsum(padded)
    offs = ends - padded
    keep = pos < CAP
    dest = jnp.where(keep, offs[flat_e] + pos, PAD - 1)
    tok = (jnp.arange(T * K, dtype=jnp.int32) // K)
    src = jnp.zeros((PAD,), jnp.int32).at[dest].set(tok)
    wflat = wts.reshape(-1) * keep.astype(jnp.float32)
    w_rows = jnp.zeros((PAD, 1), jnp.float32).at[dest, 0].set(wflat)
    w_rows = w_rows.at[PAD - 1, 0].set(0.0)
    tile_start = jnp.arange(NP, dtype=jnp.int32) * TILE
    te = jnp.sum((tile_start[:, None] >= ends[None, :]).astype(jnp.int32),
                 axis=1)
    te = jnp.minimum(te, E - 1)
    return dest, src, w_rows, te


def kernel(x, gate_w, w1, w3, w2, sw1, sw3, sw2):
    idx, wts = _gate(x, gate_w)
    dest, src, w_rows, te = _routing_metadata(idx, wts)
    xs = _dispatch_gather(x, src)
    outbuf = _ffn(te, xs, w1, w3, w2, w_rows)
    g = _combine_gather(outbuf, dest)
    y = _shared_combine(x, sw1, sw3, sw2, g.reshape(T, K * D))
    return y


# speedup vs baseline: 1.2352x; 1.0755x over previous
import functools

import jax
import jax.numpy as jnp
from jax import lax
from jax.experimental import pallas as pl
from jax.experimental.pallas import tpu as pltpu
from jax.experimental.pallas import tpu_sc as plsc

T = 4096
D = 2048
H = 1024
E = 16
K = 2
SH = 1024
CAP = (T * K // E) * 2
TILE = 256
NP = (T * K + E * (TILE - 1) + TILE - 1) // TILE
PAD = NP * TILE
GATE_TT = 512
SHARED_TT = 512
NW = 32
SC_CHUNK = 16


def _gate_body(x_ref, gw_ref, idx_ref, w_ref):
    xb = x_ref[...]
    logits = lax.dot_general(xb, gw_ref[...], (((1,), (1,)), ((), ())),
                             preferred_element_type=jnp.float32)
    m = jnp.max(logits, axis=1, keepdims=True)
    ex = jnp.exp(logits - m)
    probs = ex / jnp.sum(ex, axis=1, keepdims=True)
    lanes = lax.broadcasted_iota(jnp.int32, logits.shape, 1)
    i1 = jnp.min(jnp.where(logits == m, lanes, E), axis=1, keepdims=True)
    l2 = jnp.where(lanes == i1, -jnp.inf, logits)
    m2 = jnp.max(l2, axis=1, keepdims=True)
    i2 = jnp.min(jnp.where(l2 == m2, lanes, E), axis=1, keepdims=True)
    w1v = jnp.sum(jnp.where(lanes == i1, probs, 0.0), axis=1, keepdims=True)
    w2v = jnp.sum(jnp.where(lanes == i2, probs, 0.0), axis=1, keepdims=True)
    idx_ref[...] = jnp.concatenate([i1, i2], axis=1)
    w_ref[...] = jnp.concatenate([w1v, w2v], axis=1)


def _gate(x, gate_w):
    return pl.pallas_call(
        _gate_body,
        grid=(T // GATE_TT,),
        in_specs=[
            pl.BlockSpec((GATE_TT, D), lambda i: (i, 0)),
            pl.BlockSpec((E, D), lambda i: (0, 0)),
        ],
        out_specs=[
            pl.BlockSpec((GATE_TT, K), lambda i: (i, 0)),
            pl.BlockSpec((GATE_TT, K), lambda i: (i, 0)),
        ],
        out_shape=[
            jax.ShapeDtypeStruct((T, K), jnp.int32),
            jax.ShapeDtypeStruct((T, K), jnp.float32),
        ],
    )(x, gate_w)


@functools.lru_cache(maxsize=None)
def _make_sc_gather(n_rows):
    n_chunks = n_rows // (NW * SC_CHUNK)
    mesh = plsc.VectorSubcoreMesh(core_axis_name="c", subcore_axis_name="s")

    @functools.partial(
        pl.kernel, mesh=mesh,
        out_type=jax.ShapeDtypeStruct((n_rows, D), jnp.float32),
        scratch_types=[
            pltpu.VMEM((n_rows,), jnp.int32),
            pltpu.VMEM((SC_CHUNK, D), jnp.float32),
            pltpu.VMEM((SC_CHUNK, D), jnp.float32),
            pltpu.SemaphoreType.DMA,
            pltpu.SemaphoreType.DMA,
        ],
    )
    def k(table_hbm, idx_hbm, out_hbm, idx_v, rows0, rows1, sem0, sem1):
        wid = lax.axis_index("s") * 2 + lax.axis_index("c")
        pltpu.sync_copy(idx_hbm, idx_v)
        bufs = (rows0, rows1)
        sems = (sem0, sem1)

        def row0(c):
            return (wid + c * NW) * SC_CHUNK

        def gather(c):
            return pltpu.async_copy(
                table_hbm.at[idx_v.at[pl.ds(row0(c), SC_CHUNK)]],
                bufs[c % 2], sems[c % 2])

        handles = {0: gather(0)}
        for c in range(n_chunks):
            if c + 1 < n_chunks:
                handles[c + 1] = gather(c + 1)
            handles[c].wait()
            pltpu.sync_copy(bufs[c % 2],
                            out_hbm.at[pl.ds(row0(c), SC_CHUNK)])
    return k


def _dispatch_gather(table, idx):
    return _make_sc_gather(PAD)(table, idx)


def _combine_gather(table, idx):
    return _make_sc_gather(T * K)(table, idx)


def _ffn_body(te_ref, xs_ref, w1_ref, w3_ref, w2_ref, wr_ref, out_ref):
    del te_ref
    xb = xs_ref[...]
    h = lax.dot_general(xb, w1_ref[0], (((1,), (1,)), ((), ())),
                        preferred_element_type=jnp.float32)
    u = lax.dot_general(xb, w3_ref[0], (((1,), (1,)), ((), ())),
                        preferred_element_type=jnp.float32)
    act = h * jax.nn.sigmoid(h) * u
    o = lax.dot_general(act, w2_ref[0], (((1,), (1,)), ((), ())),
                        preferred_element_type=jnp.float32)
    out_ref[...] = o * wr_ref[...]


def _ffn(tile_expert, xs, w1, w3, w2, w_rows):
    grid_spec = pltpu.PrefetchScalarGridSpec(
        num_scalar_prefetch=1,
        grid=(NP,),
        in_specs=[
            pl.BlockSpec((TILE, D), lambda i, te: (i, 0)),
            pl.BlockSpec((1, H, D), lambda i, te: (te[i], 0, 0)),
            pl.BlockSpec((1, H, D), lambda i, te: (te[i], 0, 0)),
            pl.BlockSpec((1, D, H), lambda i, te: (te[i], 0, 0)),
            pl.BlockSpec((TILE, 1), lambda i, te: (i, 0)),
        ],
        out_specs=pl.BlockSpec((TILE, D), lambda i, te: (i, 0)),
    )
    return pl.pallas_call(
        _ffn_body,
        grid_spec=grid_spec,
        out_shape=jax.ShapeDtypeStruct((PAD, D), jnp.float32),
    )(tile_expert, xs, w1, w3, w2, w_rows)


def _shared_body(x_ref, sw1_ref, sw3_ref, sw2_ref, g_ref, out_ref):
    xb = x_ref[...]
    h = lax.dot_general(xb, sw1_ref[...], (((1,), (1,)), ((), ())),
                        preferred_element_type=jnp.float32)
    u = lax.dot_general(xb, sw3_ref[...], (((1,), (1,)), ((), ())),
                        preferred_element_type=jnp.float32)
    act = h * jax.nn.sigmoid(h) * u
    y = lax.dot_general(act, sw2_ref[...], (((1,), (1,)), ((), ())),
                        preferred_element_type=jnp.float32)
    g = g_ref[...].reshape(SHARED_TT, K, D)
    out_ref[...] = y + g[:, 0, :] + g[:, 1, :]


def _shared_combine(x, sw1, sw3, sw2, g2):
    return pl.pallas_call(
        _shared_body,
        grid=(T // SHARED_TT,),
        in_specs=[
            pl.BlockSpec((SHARED_TT, D), lambda i: (i, 0)),
            pl.BlockSpec((SH, D), lambda i: (0, 0)),
            pl.BlockSpec((SH, D), lambda i: (0, 0)),
            pl.BlockSpec((D, SH), lambda i: (0, 0)),
            pl.BlockSpec((K * SHARED_TT, D), lambda i: (i, 0)),
        ],
        out_specs=pl.BlockSpec((SHARED_TT, D), lambda i: (i, 0)),
        out_shape=jax.ShapeDtypeStruct((T, D), jnp.float32),
    )(x, sw1, sw3, sw2, g2)


def _routing_metadata(idx, wts):
    flat_e = idx.reshape(-1)
    oh = (flat_e[:, None] == jnp.arange(E, dtype=jnp.int32)[None, :])
    ohi = oh.astype(jnp.int32)
    pos = jnp.sum(jnp.cumsum(ohi, axis=0) * ohi, axis=1) - 1
    counts = jnp.sum(ohi, axis=0)
    kept = jnp.minimum(counts, CAP)
    padded = ((kept + TILE - 1) // TILE) * TILE
    ends = jnp.cumsum(padded)
    offs = ends - padded
    keep = pos < CAP
    dest = jnp.where(keep, offs[flat_e] + pos, PAD - 1)
    tok = (jnp.arange(T * K, dtype=jnp.int32) // K)
    wflat = wts.reshape(-1) * keep.astype(jnp.float32)
    packed = jnp.stack([tok, lax.bitcast_convert_type(wflat, jnp.int32)],
                       axis=1)
    inv = jnp.zeros((PAD, 2), jnp.int32).at[dest].set(packed)
    src = inv[:, 0]
    w_rows = lax.bitcast_convert_type(inv[:, 1:2], jnp.float32)
    tile_start = jnp.arange(NP, dtype=jnp.int32) * TILE
    te = jnp.sum((tile_start[:, None] >= ends[None, :]).astype(jnp.int32),
                 axis=1)
    te = jnp.minimum(te, E - 1)
    return dest, src, w_rows, te


def kernel(x, gate_w, w1, w3, w2, sw1, sw3, sw2):
    idx, wts = _gate(x, gate_w)
    dest, src, w_rows, te = _routing_metadata(idx, wts)
    xs = _dispatch_gather(x, src)
    outbuf = _ffn(te, xs, w1, w3, w2, w_rows)
    g = _combine_gather(outbuf, dest)
    y = _shared_combine(x, sw1, sw3, sw2, g)
    return y


# speedup vs baseline: 1.7061x; 1.3812x over previous
import functools

import jax
import jax.numpy as jnp
from jax import lax
from jax.experimental import pallas as pl
from jax.experimental.pallas import tpu as pltpu
from jax.experimental.pallas import tpu_sc as plsc

T = 4096
D = 2048
H = 1024
E = 16
K = 2
SH = 1024
CAP = (T * K // E) * 2
TILE = 256
NP = (T * K + E * (TILE - 1) + TILE - 1) // TILE
PAD = NP * TILE
GATE_TT = 512
SHARED_TT = 512
NW = 32
SC_CHUNK = 16


def _gate_body(x_ref, gw_ref, idx_ref, w_ref):
    xb = x_ref[...]
    logits = lax.dot_general(xb, gw_ref[...], (((1,), (1,)), ((), ())),
                             preferred_element_type=jnp.float32)
    m = jnp.max(logits, axis=1, keepdims=True)
    ex = jnp.exp(logits - m)
    probs = ex / jnp.sum(ex, axis=1, keepdims=True)
    lanes = lax.broadcasted_iota(jnp.int32, logits.shape, 1)
    i1 = jnp.min(jnp.where(logits == m, lanes, E), axis=1, keepdims=True)
    l2 = jnp.where(lanes == i1, -jnp.inf, logits)
    m2 = jnp.max(l2, axis=1, keepdims=True)
    i2 = jnp.min(jnp.where(l2 == m2, lanes, E), axis=1, keepdims=True)
    w1v = jnp.sum(jnp.where(lanes == i1, probs, 0.0), axis=1, keepdims=True)
    w2v = jnp.sum(jnp.where(lanes == i2, probs, 0.0), axis=1, keepdims=True)
    idx_ref[...] = jnp.concatenate([i1, i2], axis=1)
    w_ref[...] = jnp.concatenate([w1v, w2v], axis=1)


def _gate(x, gate_w):
    return pl.pallas_call(
        _gate_body,
        grid=(T // GATE_TT,),
        in_specs=[
            pl.BlockSpec((GATE_TT, D), lambda i: (i, 0)),
            pl.BlockSpec((E, D), lambda i: (0, 0)),
        ],
        out_specs=[
            pl.BlockSpec((GATE_TT, K), lambda i: (i, 0)),
            pl.BlockSpec((GATE_TT, K), lambda i: (i, 0)),
        ],
        out_shape=[
            jax.ShapeDtypeStruct((T, K), jnp.int32),
            jax.ShapeDtypeStruct((T, K), jnp.float32),
        ],
    )(x, gate_w)


@functools.lru_cache(maxsize=None)
def _make_sc_gather(n_rows):
    n_chunks = n_rows // (NW * SC_CHUNK)
    mesh = plsc.VectorSubcoreMesh(core_axis_name="c", subcore_axis_name="s")

    @functools.partial(
        pl.kernel, mesh=mesh,
        out_type=jax.ShapeDtypeStruct((n_rows, D), jnp.float32),
        scratch_types=[
            pltpu.VMEM((n_rows,), jnp.int32),
            pltpu.VMEM((SC_CHUNK, D), jnp.float32),
            pltpu.VMEM((SC_CHUNK, D), jnp.float32),
            pltpu.SemaphoreType.DMA,
            pltpu.SemaphoreType.DMA,
        ],
    )
    def k(table_hbm, idx_hbm, out_hbm, idx_v, rows0, rows1, sem0, sem1):
        wid = lax.axis_index("s") * 2 + lax.axis_index("c")
        pltpu.sync_copy(idx_hbm, idx_v)
        bufs = (rows0, rows1)
        sems = (sem0, sem1)

        def row0(c):
            return (wid + c * NW) * SC_CHUNK

        def gather(c):
            return pltpu.async_copy(
                table_hbm.at[idx_v.at[pl.ds(row0(c), SC_CHUNK)]],
                bufs[c % 2], sems[c % 2])

        handles = {0: gather(0)}
        for c in range(n_chunks):
            if c + 1 < n_chunks:
                handles[c + 1] = gather(c + 1)
            handles[c].wait()
            pltpu.sync_copy(bufs[c % 2],
                            out_hbm.at[pl.ds(row0(c), SC_CHUNK)])
    return k


def _dispatch_gather(table, idx):
    return _make_sc_gather(PAD)(table, idx)


def _combine_gather(table, idx):
    return _make_sc_gather(T * K)(table, idx)


def _ffn_body(te_ref, xs_ref, w1_ref, w3_ref, w2_ref, wr_ref, out_ref):
    del te_ref
    xb = xs_ref[...]
    h = lax.dot_general(xb, w1_ref[0], (((1,), (1,)), ((), ())),
                        preferred_element_type=jnp.float32)
    u = lax.dot_general(xb, w3_ref[0], (((1,), (1,)), ((), ())),
                        preferred_element_type=jnp.float32)
    act = h * jax.nn.sigmoid(h) * u
    o = lax.dot_general(act, w2_ref[0], (((1,), (1,)), ((), ())),
                        preferred_element_type=jnp.float32)
    out_ref[...] = o * wr_ref[...]


def _ffn(tile_expert, xs, w1, w3, w2, w_rows):
    grid_spec = pltpu.PrefetchScalarGridSpec(
        num_scalar_prefetch=1,
        grid=(NP,),
        in_specs=[
            pl.BlockSpec((TILE, D), lambda i, te: (i, 0)),
            pl.BlockSpec((1, H, D), lambda i, te: (te[i], 0, 0)),
            pl.BlockSpec((1, H, D), lambda i, te: (te[i], 0, 0)),
            pl.BlockSpec((1, D, H), lambda i, te: (te[i], 0, 0)),
            pl.BlockSpec((TILE, 1), lambda i, te: (i, 0)),
        ],
        out_specs=pl.BlockSpec((TILE, D), lambda i, te: (i, 0)),
    )
    return pl.pallas_call(
        _ffn_body,
        grid_spec=grid_spec,
        out_shape=jax.ShapeDtypeStruct((PAD, D), jnp.float32),
    )(tile_expert, xs, w1, w3, w2, w_rows)


def _shared_body(x_ref, sw1_ref, sw3_ref, sw2_ref, g_ref, out_ref):
    xb = x_ref[...]
    h = lax.dot_general(xb, sw1_ref[...], (((1,), (1,)), ((), ())),
                        preferred_element_type=jnp.float32)
    u = lax.dot_general(xb, sw3_ref[...], (((1,), (1,)), ((), ())),
                        preferred_element_type=jnp.float32)
    act = h * jax.nn.sigmoid(h) * u
    y = lax.dot_general(act, sw2_ref[...], (((1,), (1,)), ((), ())),
                        preferred_element_type=jnp.float32)
    g = g_ref[...].reshape(SHARED_TT, K, D)
    out_ref[...] = y + g[:, 0, :] + g[:, 1, :]


def _shared_combine(x, sw1, sw3, sw2, g2):
    return pl.pallas_call(
        _shared_body,
        grid=(T // SHARED_TT,),
        in_specs=[
            pl.BlockSpec((SHARED_TT, D), lambda i: (i, 0)),
            pl.BlockSpec((SH, D), lambda i: (0, 0)),
            pl.BlockSpec((SH, D), lambda i: (0, 0)),
            pl.BlockSpec((D, SH), lambda i: (0, 0)),
            pl.BlockSpec((K * SHARED_TT, D), lambda i: (i, 0)),
        ],
        out_specs=pl.BlockSpec((SHARED_TT, D), lambda i: (i, 0)),
        out_shape=jax.ShapeDtypeStruct((T, D), jnp.float32),
    )(x, sw1, sw3, sw2, g2)


def _routing_metadata(idx, wts):
    flat_e = idx.reshape(-1)
    oh = (flat_e[:, None] == jnp.arange(E, dtype=jnp.int32)[None, :])
    ohi = oh.astype(jnp.int32)
    pos = jnp.sum(jnp.cumsum(ohi, axis=0) * ohi, axis=1) - 1
    counts = jnp.sum(ohi, axis=0)
    kept = jnp.minimum(counts, CAP)
    padded = ((kept + TILE - 1) // TILE) * TILE
    ends = jnp.cumsum(padded)
    offs = ends - padded
    keep = pos < CAP
    dest = jnp.where(keep, offs[flat_e] + pos, PAD - 1)
    tok = (jnp.arange(T * K, dtype=jnp.int32) // K)
    wflat = wts.reshape(-1) * keep.astype(jnp.float32)
    packed = jnp.stack([tok, lax.bitcast_convert_type(wflat, jnp.int32)],
                       axis=1)
    base_src = jnp.arange(PAD, dtype=jnp.int32) & (T - 1)
    inv = jnp.stack([base_src, jnp.zeros((PAD,), jnp.int32)],
                    axis=1).at[dest].set(packed)
    src = inv[:, 0]
    w_rows = lax.bitcast_convert_type(inv[:, 1:2], jnp.float32)
    tile_start = jnp.arange(NP, dtype=jnp.int32) * TILE
    te = jnp.sum((tile_start[:, None] >= ends[None, :]).astype(jnp.int32),
                 axis=1)
    te = jnp.minimum(te, E - 1)
    return dest, src, w_rows, te


def kernel(x, gate_w, w1, w3, w2, sw1, sw3, sw2):
    idx, wts = _gate(x, gate_w)
    dest, src, w_rows, te = _routing_metadata(idx, wts)
    xs = _dispatch_gather(x, src)
    outbuf = _ffn(te, xs, w1, w3, w2, w_rows)
    g = _combine_gather(outbuf, dest)
    y = _shared_combine(x, sw1, sw3, sw2, g)
    return y


# speedup vs baseline: 1.7665x; 1.0354x over previous
import functools

import jax
import jax.numpy as jnp
from jax import lax
from jax.experimental import pallas as pl
from jax.experimental.pallas import tpu as pltpu
from jax.experimental.pallas import tpu_sc as plsc

T = 4096
D = 2048
H = 1024
E = 16
K = 2
SH = 1024
CAP = (T * K // E) * 2
TILE = 256
NP = (T * K + E * (TILE - 1) + TILE - 1) // TILE
PAD = NP * TILE
GATE_TT = 512
SHARED_TT = 512
NW = 32
SC_CHUNK = 16


def _gate_body(x_ref, gw_ref, idx_ref, w_ref):
    xb = x_ref[...]
    logits = lax.dot_general(xb, gw_ref[...], (((1,), (1,)), ((), ())),
                             preferred_element_type=jnp.float32)
    m = jnp.max(logits, axis=1, keepdims=True)
    ex = jnp.exp(logits - m)
    probs = ex / jnp.sum(ex, axis=1, keepdims=True)
    lanes = lax.broadcasted_iota(jnp.int32, logits.shape, 1)
    i1 = jnp.min(jnp.where(logits == m, lanes, E), axis=1, keepdims=True)
    l2 = jnp.where(lanes == i1, -jnp.inf, logits)
    m2 = jnp.max(l2, axis=1, keepdims=True)
    i2 = jnp.min(jnp.where(l2 == m2, lanes, E), axis=1, keepdims=True)
    w1v = jnp.sum(jnp.where(lanes == i1, probs, 0.0), axis=1, keepdims=True)
    w2v = jnp.sum(jnp.where(lanes == i2, probs, 0.0), axis=1, keepdims=True)
    idx_ref[...] = jnp.concatenate([i1, i2], axis=1)
    w_ref[...] = jnp.concatenate([w1v, w2v], axis=1)


def _gate(x, gate_w):
    return pl.pallas_call(
        _gate_body,
        grid=(T // GATE_TT,),
        in_specs=[
            pl.BlockSpec((GATE_TT, D), lambda i: (i, 0)),
            pl.BlockSpec((E, D), lambda i: (0, 0)),
        ],
        out_specs=[
            pl.BlockSpec((GATE_TT, K), lambda i: (i, 0)),
            pl.BlockSpec((GATE_TT, K), lambda i: (i, 0)),
        ],
        out_shape=[
            jax.ShapeDtypeStruct((T, K), jnp.int32),
            jax.ShapeDtypeStruct((T, K), jnp.float32),
        ],
    )(x, gate_w)


@functools.lru_cache(maxsize=None)
def _make_sc_gather(n_rows):
    n_chunks = n_rows // (NW * SC_CHUNK)
    mesh = plsc.VectorSubcoreMesh(core_axis_name="c", subcore_axis_name="s")

    @functools.partial(
        pl.kernel, mesh=mesh,
        out_type=jax.ShapeDtypeStruct((n_rows, D), jnp.float32),
        scratch_types=[
            pltpu.VMEM((n_rows,), jnp.int32),
            pltpu.VMEM((SC_CHUNK, D), jnp.float32),
            pltpu.VMEM((SC_CHUNK, D), jnp.float32),
            pltpu.SemaphoreType.DMA,
            pltpu.SemaphoreType.DMA,
        ],
    )
    def k(table_hbm, idx_hbm, out_hbm, idx_v, rows0, rows1, sem0, sem1):
        wid = lax.axis_index("s") * 2 + lax.axis_index("c")
        pltpu.sync_copy(idx_hbm, idx_v)
        bufs = (rows0, rows1)
        sems = (sem0, sem1)

        def row0(c):
            return (wid + c * NW) * SC_CHUNK

        def gather(c):
            return pltpu.async_copy(
                table_hbm.at[idx_v.at[pl.ds(row0(c), SC_CHUNK)]],
                bufs[c % 2], sems[c % 2])

        handles = {0: gather(0)}
        for c in range(n_chunks):
            if c + 1 < n_chunks:
                handles[c + 1] = gather(c + 1)
            handles[c].wait()
            pltpu.sync_copy(bufs[c % 2],
                            out_hbm.at[pl.ds(row0(c), SC_CHUNK)])
    return k


def _dispatch_gather(table, idx):
    return _make_sc_gather(PAD)(table, idx)


def _combine_gather(table, idx):
    return _make_sc_gather(T * K)(table, idx)


def _ffn_body(te_ref, valid_ref, xs_ref, w1_ref, w3_ref, w2_ref, wr_ref,
              out_ref):
    del te_ref
    i = pl.program_id(0)

    @pl.when(valid_ref[i] == 1)
    def _compute():
        xb = xs_ref[...]
        h = lax.dot_general(xb, w1_ref[0], (((1,), (1,)), ((), ())),
                            preferred_element_type=jnp.float32)
        u = lax.dot_general(xb, w3_ref[0], (((1,), (1,)), ((), ())),
                            preferred_element_type=jnp.float32)
        act = h * jax.nn.sigmoid(h) * u
        o = lax.dot_general(act, w2_ref[0], (((1,), (1,)), ((), ())),
                            preferred_element_type=jnp.float32)
        out_ref[...] = o * wr_ref[...]

    @pl.when(valid_ref[i] == 0)
    def _zero():
        out_ref[...] = jnp.zeros_like(out_ref)


def _ffn(tile_expert, tile_valid, xs, w1, w3, w2, w_rows):
    grid_spec = pltpu.PrefetchScalarGridSpec(
        num_scalar_prefetch=2,
        grid=(NP,),
        in_specs=[
            pl.BlockSpec((TILE, D), lambda i, te, v: (i, 0)),
            pl.BlockSpec((1, H, D), lambda i, te, v: (te[i], 0, 0)),
            pl.BlockSpec((1, H, D), lambda i, te, v: (te[i], 0, 0)),
            pl.BlockSpec((1, D, H), lambda i, te, v: (te[i], 0, 0)),
            pl.BlockSpec((TILE, 1), lambda i, te, v: (i, 0)),
        ],
        out_specs=pl.BlockSpec((TILE, D), lambda i, te, v: (i, 0)),
    )
    return pl.pallas_call(
        _ffn_body,
        grid_spec=grid_spec,
        out_shape=jax.ShapeDtypeStruct((PAD, D), jnp.float32),
    )(tile_expert, tile_valid, xs, w1, w3, w2, w_rows)


def _shared_body(x_ref, sw1_ref, sw3_ref, sw2_ref, g_ref, out_ref):
    xb = x_ref[...]
    h = lax.dot_general(xb, sw1_ref[...], (((1,), (1,)), ((), ())),
                        preferred_element_type=jnp.float32)
    u = lax.dot_general(xb, sw3_ref[...], (((1,), (1,)), ((), ())),
                        preferred_element_type=jnp.float32)
    act = h * jax.nn.sigmoid(h) * u
    y = lax.dot_general(act, sw2_ref[...], (((1,), (1,)), ((), ())),
                        preferred_element_type=jnp.float32)
    g = g_ref[...].reshape(SHARED_TT, K, D)
    out_ref[...] = y + g[:, 0, :] + g[:, 1, :]


def _shared_combine(x, sw1, sw3, sw2, g2):
    return pl.pallas_call(
        _shared_body,
        grid=(T // SHARED_TT,),
        in_specs=[
            pl.BlockSpec((SHARED_TT, D), lambda i: (i, 0)),
            pl.BlockSpec((SH, D), lambda i: (0, 0)),
            pl.BlockSpec((SH, D), lambda i: (0, 0)),
            pl.BlockSpec((D, SH), lambda i: (0, 0)),
            pl.BlockSpec((K * SHARED_TT, D), lambda i: (i, 0)),
        ],
        out_specs=pl.BlockSpec((SHARED_TT, D), lambda i: (i, 0)),
        out_shape=jax.ShapeDtypeStruct((T, D), jnp.float32),
    )(x, sw1, sw3, sw2, g2)


def _routing_metadata(idx, wts):
    flat_e = idx.reshape(-1)
    oh = (flat_e[:, None] == jnp.arange(E, dtype=jnp.int32)[None, :])
    ohi = oh.astype(jnp.int32)
    pos = jnp.sum(jnp.cumsum(ohi, axis=0) * ohi, axis=1) - 1
    counts = jnp.sum(ohi, axis=0)
    kept = jnp.minimum(counts, CAP)
    padded = ((kept + TILE - 1) // TILE) * TILE
    ends = jnp.cumsum(padded)
    offs = ends - padded
    keep = pos < CAP
    dest = jnp.where(keep, offs[flat_e] + pos, PAD - 1)
    tok = (jnp.arange(T * K, dtype=jnp.int32) // K)
    wflat = wts.reshape(-1) * keep.astype(jnp.float32)
    packed = jnp.stack([tok, lax.bitcast_convert_type(wflat, jnp.int32)],
                       axis=1)
    base_src = jnp.arange(PAD, dtype=jnp.int32) & (T - 1)
    inv = jnp.stack([base_src, jnp.zeros((PAD,), jnp.int32)],
                    axis=1).at[dest].set(packed)
    src = inv[:, 0]
    w_rows = lax.bitcast_convert_type(inv[:, 1:2], jnp.float32)
    tile_start = jnp.arange(NP, dtype=jnp.int32) * TILE
    te = jnp.sum((tile_start[:, None] >= ends[None, :]).astype(jnp.int32),
                 axis=1)
    te = jnp.minimum(te, E - 1)
    tv = (tile_start < ends[E - 1]).astype(jnp.int32)
    return dest, src, w_rows, te, tv


def kernel(x, gate_w, w1, w3, w2, sw1, sw3, sw2):
    idx, wts = _gate(x, gate_w)
    dest, src, w_rows, te, tv = _routing_metadata(idx, wts)
    xs = _dispatch_gather(x, src)
    outbuf = _ffn(te, tv, xs, w1, w3, w2, w_rows)
    g = _combine_gather(outbuf, dest)
    y = _shared_combine(x, sw1, sw3, sw2, g)
    return y


# speedup vs baseline: 1.8491x; 1.0468x over previous
import functools

import jax
import jax.numpy as jnp
from jax import lax
from jax.experimental import pallas as pl
from jax.experimental.pallas import tpu as pltpu
from jax.experimental.pallas import tpu_sc as plsc

T = 4096
D = 2048
H = 1024
E = 16
K = 2
SH = 1024
CAP = (T * K // E) * 2
TILE = 256
NP = (T * K + E * (TILE - 1) + TILE - 1) // TILE
PAD = NP * TILE
GATE_TT = 512
SHARED_TT = 512
NW = 32
SC_CHUNK = 16


def _gate_body(x_ref, gw_ref, idx_ref, w_ref):
    xb = x_ref[...]
    logits = lax.dot_general(xb, gw_ref[...], (((1,), (1,)), ((), ())),
                             preferred_element_type=jnp.float32)
    m = jnp.max(logits, axis=1, keepdims=True)
    ex = jnp.exp(logits - m)
    probs = ex / jnp.sum(ex, axis=1, keepdims=True)
    lanes = lax.broadcasted_iota(jnp.int32, logits.shape, 1)
    i1 = jnp.min(jnp.where(logits == m, lanes, E), axis=1, keepdims=True)
    l2 = jnp.where(lanes == i1, -jnp.inf, logits)
    m2 = jnp.max(l2, axis=1, keepdims=True)
    i2 = jnp.min(jnp.where(l2 == m2, lanes, E), axis=1, keepdims=True)
    w1v = jnp.sum(jnp.where(lanes == i1, probs, 0.0), axis=1, keepdims=True)
    w2v = jnp.sum(jnp.where(lanes == i2, probs, 0.0), axis=1, keepdims=True)
    idx_ref[...] = jnp.concatenate([i1, i2], axis=1)
    w_ref[...] = jnp.concatenate([w1v, w2v], axis=1)


def _gate(x, gate_w):
    return pl.pallas_call(
        _gate_body,
        grid=(T // GATE_TT,),
        in_specs=[
            pl.BlockSpec((GATE_TT, D), lambda i: (i, 0)),
            pl.BlockSpec((E, D), lambda i: (0, 0)),
        ],
        out_specs=[
            pl.BlockSpec((GATE_TT, K), lambda i: (i, 0)),
            pl.BlockSpec((GATE_TT, K), lambda i: (i, 0)),
        ],
        out_shape=[
            jax.ShapeDtypeStruct((T, K), jnp.int32),
            jax.ShapeDtypeStruct((T, K), jnp.float32),
        ],
    )(x, gate_w)


@functools.lru_cache(maxsize=None)
def _make_sc_gather(n_rows):
    n_chunks = n_rows // (NW * SC_CHUNK)
    mesh = plsc.VectorSubcoreMesh(core_axis_name="c", subcore_axis_name="s")

    @functools.partial(
        pl.kernel, mesh=mesh,
        out_type=jax.ShapeDtypeStruct((n_rows, D), jnp.float32),
        scratch_types=[
            pltpu.VMEM((n_rows,), jnp.int32),
            pltpu.VMEM((SC_CHUNK, D), jnp.float32),
            pltpu.VMEM((SC_CHUNK, D), jnp.float32),
            pltpu.SemaphoreType.DMA,
            pltpu.SemaphoreType.DMA,
        ],
    )
    def k(table_hbm, idx_hbm, out_hbm, idx_v, rows0, rows1, sem0, sem1):
        wid = lax.axis_index("s") * 2 + lax.axis_index("c")
        pltpu.sync_copy(idx_hbm, idx_v)
        bufs = (rows0, rows1)
        sems = (sem0, sem1)

        def row0(c):
            return (wid + c * NW) * SC_CHUNK

        def gather(c):
            return pltpu.async_copy(
                table_hbm.at[idx_v.at[pl.ds(row0(c), SC_CHUNK)]],
                bufs[c % 2], sems[c % 2])

        handles = {0: gather(0)}
        for c in range(n_chunks):
            if c + 1 < n_chunks:
                handles[c + 1] = gather(c + 1)
            handles[c].wait()
            pltpu.sync_copy(bufs[c % 2],
                            out_hbm.at[pl.ds(row0(c), SC_CHUNK)])
    return k


def _dispatch_gather(table, idx):
    return _make_sc_gather(PAD)(table, idx)


def _combine_gather(table, idx):
    return _make_sc_gather(T * K)(table, idx)


def _ffn_body(te_ref, valid_ref, xs_ref, w1_ref, w3_ref, w2_ref, wr_ref,
              out_ref):
    del te_ref
    i = pl.program_id(0)

    @pl.when(valid_ref[i] == 1)
    def _compute():
        xb = xs_ref[...]
        h = lax.dot_general(xb, w1_ref[0], (((1,), (1,)), ((), ())),
                            preferred_element_type=jnp.float32)
        u = lax.dot_general(xb, w3_ref[0], (((1,), (1,)), ((), ())),
                            preferred_element_type=jnp.float32)
        act = h * jax.nn.sigmoid(h) * u
        o = lax.dot_general(act, w2_ref[0], (((1,), (1,)), ((), ())),
                            preferred_element_type=jnp.float32)
        out_ref[...] = o * wr_ref[...]

    @pl.when(valid_ref[i] == 0)
    def _zero():
        out_ref[...] = jnp.zeros_like(out_ref)


def _ffn(tile_expert, tile_valid, xs, w1, w3, w2, w_rows):
    grid_spec = pltpu.PrefetchScalarGridSpec(
        num_scalar_prefetch=2,
        grid=(NP,),
        in_specs=[
            pl.BlockSpec((TILE, D), lambda i, te, v: (i, 0)),
            pl.BlockSpec((1, H, D), lambda i, te, v: (te[i], 0, 0)),
            pl.BlockSpec((1, H, D), lambda i, te, v: (te[i], 0, 0)),
            pl.BlockSpec((1, D, H), lambda i, te, v: (te[i], 0, 0)),
            pl.BlockSpec((TILE, 1), lambda i, te, v: (i, 0)),
        ],
        out_specs=pl.BlockSpec((TILE, D), lambda i, te, v: (i, 0)),
    )
    return pl.pallas_call(
        _ffn_body,
        grid_spec=grid_spec,
        out_shape=jax.ShapeDtypeStruct((PAD, D), jnp.float32),
    )(tile_expert, tile_valid, xs, w1, w3, w2, w_rows)


def _shared_body(x_ref, sw1_ref, sw3_ref, sw2_ref, out_ref):
    xb = x_ref[...]
    h = lax.dot_general(xb, sw1_ref[...], (((1,), (1,)), ((), ())),
                        preferred_element_type=jnp.float32)
    u = lax.dot_general(xb, sw3_ref[...], (((1,), (1,)), ((), ())),
                        preferred_element_type=jnp.float32)
    act = h * jax.nn.sigmoid(h) * u
    y = lax.dot_general(act, sw2_ref[...], (((1,), (1,)), ((), ())),
                        preferred_element_type=jnp.float32)
    out_ref[...] = y


def _shared_ffn(x, sw1, sw3, sw2):
    return pl.pallas_call(
        _shared_body,
        grid=(T // SHARED_TT,),
        in_specs=[
            pl.BlockSpec((SHARED_TT, D), lambda i: (i, 0)),
            pl.BlockSpec((SH, D), lambda i: (0, 0)),
            pl.BlockSpec((SH, D), lambda i: (0, 0)),
            pl.BlockSpec((D, SH), lambda i: (0, 0)),
        ],
        out_specs=pl.BlockSpec((SHARED_TT, D), lambda i: (i, 0)),
        out_shape=jax.ShapeDtypeStruct((T, D), jnp.float32),
    )(x, sw1, sw3, sw2)


def _add_body(sy_ref, g_ref, out_ref):
    g = g_ref[...].reshape(SHARED_TT, K, D)
    out_ref[...] = sy_ref[...] + g[:, 0, :] + g[:, 1, :]


def _combine_add(sy, g):
    return pl.pallas_call(
        _add_body,
        grid=(T // SHARED_TT,),
        in_specs=[
            pl.BlockSpec((SHARED_TT, D), lambda i: (i, 0)),
            pl.BlockSpec((K * SHARED_TT, D), lambda i: (i, 0)),
        ],
        out_specs=pl.BlockSpec((SHARED_TT, D), lambda i: (i, 0)),
        out_shape=jax.ShapeDtypeStruct((T, D), jnp.float32),
    )(sy, g)


def _routing_metadata(idx, wts):
    flat_e = idx.reshape(-1)
    oh = (flat_e[:, None] == jnp.arange(E, dtype=jnp.int32)[None, :])
    ohi = oh.astype(jnp.int32)
    pos = jnp.sum(jnp.cumsum(ohi, axis=0) * ohi, axis=1) - 1
    counts = jnp.sum(ohi, axis=0)
    kept = jnp.minimum(counts, CAP)
    padded = ((kept + TILE - 1) // TILE) * TILE
    ends = jnp.cumsum(padded)
    offs = ends - padded
    keep = pos < CAP
    dest = jnp.where(keep, offs[flat_e] + pos, PAD - 1)
    tok = (jnp.arange(T * K, dtype=jnp.int32) // K)
    wflat = wts.reshape(-1) * keep.astype(jnp.float32)
    packed = jnp.stack([tok, lax.bitcast_convert_type(wflat, jnp.int32)],
                       axis=1)
    base_src = jnp.arange(PAD, dtype=jnp.int32) & (T - 1)
    inv = jnp.stack([base_src, jnp.zeros((PAD,), jnp.int32)],
                    axis=1).at[dest].set(packed)
    src = inv[:, 0]
    w_rows = lax.bitcast_convert_type(inv[:, 1:2], jnp.float32)
    tile_start = jnp.arange(NP, dtype=jnp.int32) * TILE
    te = jnp.sum((tile_start[:, None] >= ends[None, :]).astype(jnp.int32),
                 axis=1)
    te = jnp.minimum(te, E - 1)
    tv = (tile_start < ends[E - 1]).astype(jnp.int32)
    return dest, src, w_rows, te, tv


def kernel(x, gate_w, w1, w3, w2, sw1, sw3, sw2):
    idx, wts = _gate(x, gate_w)
    dest, src, w_rows, te, tv = _routing_metadata(idx, wts)
    xs = _dispatch_gather(x, src)
    sy = _shared_ffn(x, sw1, sw3, sw2)
    outbuf = _ffn(te, tv, xs, w1, w3, w2, w_rows)
    g = _combine_gather(outbuf, dest)
    return _combine_add(sy, g)
